# baseline (speedup 1.0000x reference)
import functools

import jax
import jax.numpy as jnp
from jax import lax
from jax.experimental import pallas as pl
from jax.experimental.pallas import tpu as pltpu
from jax.experimental.pallas import tpu_sc as plsc

VOCAB = 1000000
EMBED = 64
LABEL = 16
B = 4096
L = 200

NUM_CORES = 2
NUM_SUBCORES = 16
NW = NUM_CORES * NUM_SUBCORES
ROWS_PER_W = B // NW
CHUNK = 100
CHUNKS_PER_ROW = L // CHUNK
IDX_ROWS_PER_W = ROWS_PER_W * CHUNKS_PER_ROW
LANES = 16
EGROUPS = EMBED // LANES
PAIR = 2 * EMBED


BAND = 128
TILES_PER_W = 244
COLS_PER_W = TILES_PER_W * BAND
TAIL0 = NW * COLS_PER_W


def _transpose_band(stage, out_buf, ncols):
    iota = lax.iota(jnp.int32, LANES)
    half = iota >> 1
    par64 = (iota & 1) << 6
    perms = [(iota + s) & 15 for s in range(LANES)]
    colbases = [par64 + (16 * g2) for g2 in range(EGROUPS)]
    nk = ncols // LANES

    @plsc.parallel_loop(0, nk, unroll=2)
    def kblock(k):
        tvec = iota + k * LANES
        rowvec = half + k * (LANES // 2)
        for g2 in range(EGROUPS):
            for s in range(LANES):
                cvec = perms[s] + (16 * g2)
                colvec = colbases[g2] + perms[s]
                v = plsc.load_gather(stage, [cvec, tvec])
                plsc.store_scatter(out_buf, [rowvec, colvec], v)


NBUF = 4


def _relayout_body(table_t, tail_t, out_hbm,
                   stage0, stage1, stage2, stage3, out0, out1, out2, out3,
                   tail_stage, sin0, sin1, sin2, sin3,
                   sout0, sout1, sout2, sout3):
    wid = lax.axis_index("s") * NUM_CORES + lax.axis_index("c")
    c_base = wid * COLS_PER_W

    stages = (stage0, stage1, stage2, stage3)
    outs = (out0, out1, out2, out3)
    sins = (sin0, sin1, sin2, sin3)
    souts = (sout0, sout1, sout2, sout3)

    def src(col0):
        return table_t.at[pl.ds(0, EMBED),
                          pl.ds(pl.multiple_of(col0, BAND), BAND)]

    def dst(row0, n):
        return out_hbm.at[pl.ds(pl.multiple_of(row0, 8), n)]

    for b in range(NBUF):
        pltpu.async_copy(src(c_base + b * BAND), stages[b], sins[b])

    def ring_body(i, carry):
        for b in range(NBUF):
            band = NBUF * i + b
            col0 = c_base + band * BAND
            pltpu.make_async_copy(src(col0), stages[b], sins[b]).wait()
            prev = col0 - NBUF * BAND

            @pl.when(band >= NBUF)
            def _():
                pltpu.make_async_copy(
                    outs[b], dst(prev // 2, BAND // 2), souts[b]).wait()

            _transpose_band(stages[b], outs[b], BAND)
            nxt = jnp.minimum(col0 + NBUF * BAND,
                              c_base + COLS_PER_W - BAND)
            pltpu.async_copy(src(nxt), stages[b], sins[b])
            pltpu.async_copy(outs[b], dst(col0 // 2, BAND // 2), souts[b])
        return carry

    lax.fori_loop(0, TILES_PER_W // NBUF, ring_body, 0)

    for b in range(NBUF):
        last = c_base + COLS_PER_W - (NBUF - b) * BAND
        pltpu.make_async_copy(src(jnp.int32(0)), stages[b], sins[b]).wait()
        pltpu.make_async_copy(
            outs[b], dst(last // 2, BAND // 2), souts[b]).wait()

    @pl.when(wid < 4)
    def _():
        col0 = TAIL0 + BAND * wid
        pltpu.sync_copy(src(col0), stage0)
        _transpose_band(stage0, out0, BAND)
        pltpu.sync_copy(out0, dst(col0 // 2, BAND // 2))

    @pl.when(wid == 4)
    def _():
        col0 = TAIL0 + 4 * BAND
        pltpu.sync_copy(tail_t, tail_stage)
        _transpose_band(tail_stage, out0, EMBED)
        pltpu.sync_copy(out0.at[pl.ds(0, 32)], dst(col0 // 2, 32))


_relayout_sc = functools.partial(
    pl.kernel,
    out_type=jax.ShapeDtypeStruct((VOCAB // 2, PAIR), jnp.float32),
    mesh=plsc.VectorSubcoreMesh(core_axis_name="c", subcore_axis_name="s"),
    scratch_types=(
        [pltpu.VMEM((EMBED, BAND), jnp.float32)] * NBUF
        + [pltpu.VMEM((BAND // 2, PAIR), jnp.float32)] * NBUF
        + [pltpu.VMEM((EMBED, EMBED), jnp.float32)]
        + [pltpu.SemaphoreType.DMA] * (2 * NBUF)
    ),
    compiler_params=pltpu.CompilerParams(use_tc_tiling_on_sc=True,
                                         needs_layout_passes=False),
)(_relayout_body)


def _reduce_chunk(rows_ref, accs):
    def red(j, accs):
        sA = list(accs[0:4]); sB = list(accs[4:8])
        mA = list(accs[8:12]); mB = list(accs[12:16])
        for g in range(EGROUPS):
            va = rows_ref[2 * j, pl.ds(g * LANES, LANES)]
            vb = rows_ref[2 * j + 1, pl.ds(g * LANES, LANES)]
            sA[g] = sA[g] + va
            sB[g] = sB[g] + vb
            mA[g] = jnp.maximum(mA[g], va)
            mB[g] = jnp.maximum(mB[g], vb)
        return tuple(sA + sB + mA + mB)

    return lax.fori_loop(0, CHUNK // 2, red, accs, unroll=4)


def _fresh_accs():
    z = jnp.zeros((LANES,), jnp.float32)
    ninf = jnp.full((LANES,), -jnp.inf, jnp.float32)
    return tuple([z] * (2 * EGROUPS) + [ninf] * (2 * EGROUPS))


PBUF = 4


def _pool_body(idx_hbm, table_hbm, out_hbm, idx_v,
               rows0, rows1, rows2, rows3, pooled_v,
               sem0, sem1, sem2, sem3):
    wid = lax.axis_index("s") * NUM_CORES + lax.axis_index("c")
    base = wid * ROWS_PER_W
    rows = (rows0, rows1, rows2, rows3)
    sems = (sem0, sem1, sem2, sem3)

    pltpu.sync_copy(idx_hbm.at[pl.ds(wid * IDX_ROWS_PER_W, IDX_ROWS_PER_W)],
                    idx_v)

    def fire(c, b):
        pltpu.async_copy(table_hbm.at[idx_v.at[c]], rows[b], sems[b])

    def wait(b):
        pltpu.make_async_copy(table_hbm.at[idx_v.at[0]], rows[b],
                              sems[b]).wait()

    for b in range(PBUF):
        fire(b, b)

    def pair_body(ee, carry):
        for p in range(2):
            e = 2 * ee + p
            b0 = 2 * p
            b1 = 2 * p + 1
            wait(b0)
            accs = _reduce_chunk(rows[b0], _fresh_accs())
            fire(jnp.minimum(2 * e + PBUF, IDX_ROWS_PER_W - 1), b0)
            wait(b1)
            accs = _reduce_chunk(rows[b1], accs)
            fire(jnp.minimum(2 * e + PBUF + 1, IDX_ROWS_PER_W - 1), b1)

            inv_l = jnp.float32(1.0 / L)
            for g in range(EGROUPS):
                s = accs[g] + accs[EGROUPS + g]
                m = jnp.maximum(accs[2 * EGROUPS + g], accs[3 * EGROUPS + g])
                pooled_v[e, pl.ds(g * LANES, LANES)] = s * inv_l + m
        return carry

    lax.fori_loop(0, ROWS_PER_W // 2, pair_body, 0)
    for b in range(PBUF):
        wait(b)
    pltpu.sync_copy(pooled_v, out_hbm.at[pl.ds(base, ROWS_PER_W)])


_pool_sc = functools.partial(
    pl.kernel,
    out_type=jax.ShapeDtypeStruct((B, EMBED), jnp.float32),
    mesh=plsc.VectorSubcoreMesh(core_axis_name="c", subcore_axis_name="s"),
    scratch_types=(
        [pltpu.VMEM((IDX_ROWS_PER_W, CHUNK), jnp.int32)]
        + [pltpu.VMEM((CHUNK, EMBED), jnp.float32)] * PBUF
        + [pltpu.VMEM((ROWS_PER_W, EMBED), jnp.float32)]
        + [pltpu.SemaphoreType.DMA] * PBUF
    ),
    compiler_params=pltpu.CompilerParams(use_tc_tiling_on_sc=False,
                                         needs_layout_passes=False),
)(_pool_body)


def _mm_body(p_ref, w_ref, b_ref, o_ref):
    o_ref[...] = (
        jnp.dot(p_ref[...], w_ref[...], preferred_element_type=jnp.float32)
        + b_ref[...])


def _project(pooled, W, b):
    return pl.pallas_call(
        _mm_body,
        out_shape=jax.ShapeDtypeStruct((B, LABEL), jnp.float32),
    )(pooled, W, b.reshape(1, LABEL))


def kernel(input, emb_table, W, b):
    idx = input.astype(jnp.int32).reshape(B * L // CHUNK, CHUNK)
    emb_t = emb_table.T
    emb2 = _relayout_sc(emb_t, lax.slice(emb_t, (0, VOCAB - EMBED),
                                         (EMBED, VOCAB)))
    table_lin = emb2.reshape(VOCAB, EMBED)
    pooled = _pool_sc(idx, table_lin)
    return _project(pooled, W, b)


# speedup vs baseline: 1.0187x; 1.0187x over previous
import functools

import jax
import jax.numpy as jnp
from jax import lax
from jax.experimental import pallas as pl
from jax.experimental.pallas import tpu as pltpu
from jax.experimental.pallas import tpu_sc as plsc

VOCAB = 1000000
EMBED = 64
LABEL = 16
B = 4096
L = 200

NUM_CORES = 2
NUM_SUBCORES = 16
NW = NUM_CORES * NUM_SUBCORES
ROWS_PER_W = B // NW
CHUNK = 100
CHUNKS_PER_ROW = L // CHUNK
IDX_ROWS_PER_W = ROWS_PER_W * CHUNKS_PER_ROW
LANES = 16
EGROUPS = EMBED // LANES
PAIR = 2 * EMBED


BAND = 128
TILES_PER_W = 244
COLS_PER_W = TILES_PER_W * BAND
TAIL0 = NW * COLS_PER_W


def _transpose_band(stage, out_buf, ncols):
    iota = lax.iota(jnp.int32, LANES)
    half = iota >> 1
    par64 = (iota & 1) << 6
    perms = [(iota + s) & 15 for s in range(LANES)]
    colbases = [par64 + (16 * g2) for g2 in range(EGROUPS)]
    nk = ncols // LANES

    @plsc.parallel_loop(0, nk, unroll=2)
    def kblock(k):
        tvec = iota + k * LANES
        rowvec = half + k * (LANES // 2)
        for g2 in range(EGROUPS):
            for s in range(LANES):
                cvec = perms[s] + (16 * g2)
                colvec = colbases[g2] + perms[s]
                v = plsc.load_gather(stage, [cvec, tvec])
                plsc.store_scatter(out_buf, [rowvec, colvec], v)


NBUF = 4


def _relayout_body(table_t, tail_t, out_hbm,
                   stage0, stage1, stage2, stage3, out0, out1, out2, out3,
                   tail_stage, sin0, sin1, sin2, sin3,
                   sout0, sout1, sout2, sout3):
    wid = lax.axis_index("s") * NUM_CORES + lax.axis_index("c")
    c_base = wid * COLS_PER_W

    stages = (stage0, stage1, stage2, stage3)
    outs = (out0, out1, out2, out3)
    sins = (sin0, sin1, sin2, sin3)
    souts = (sout0, sout1, sout2, sout3)

    def src(col0):
        return table_t.at[pl.ds(0, EMBED),
                          pl.ds(pl.multiple_of(col0, BAND), BAND)]

    def dst(row0, n):
        return out_hbm.at[pl.ds(pl.multiple_of(row0, 8), n)]

    for b in range(NBUF):
        pltpu.async_copy(src(c_base + b * BAND), stages[b], sins[b])

    def ring_body(i, carry):
        for b in range(NBUF):
            band = NBUF * i + b
            col0 = c_base + band * BAND
            pltpu.make_async_copy(src(col0), stages[b], sins[b]).wait()
            prev = col0 - NBUF * BAND

            @pl.when(band >= NBUF)
            def _():
                pltpu.make_async_copy(
                    outs[b], dst(prev // 2, BAND // 2), souts[b]).wait()

            _transpose_band(stages[b], outs[b], BAND)
            nxt = jnp.minimum(col0 + NBUF * BAND,
                              c_base + COLS_PER_W - BAND)
            pltpu.async_copy(src(nxt), stages[b], sins[b])
            pltpu.async_copy(outs[b], dst(col0 // 2, BAND // 2), souts[b])
        return carry

    lax.fori_loop(0, TILES_PER_W // NBUF, ring_body, 0)

    for b in range(NBUF):
        last = c_base + COLS_PER_W - (NBUF - b) * BAND
        pltpu.make_async_copy(src(jnp.int32(0)), stages[b], sins[b]).wait()
        pltpu.make_async_copy(
            outs[b], dst(last // 2, BAND // 2), souts[b]).wait()

    @pl.when(wid < 4)
    def _():
        col0 = TAIL0 + BAND * wid
        pltpu.sync_copy(src(col0), stage0)
        _transpose_band(stage0, out0, BAND)
        pltpu.sync_copy(out0, dst(col0 // 2, BAND // 2))

    @pl.when(wid == 4)
    def _():
        col0 = TAIL0 + 4 * BAND
        pltpu.sync_copy(tail_t, tail_stage)
        _transpose_band(tail_stage, out0, EMBED)
        pltpu.sync_copy(out0.at[pl.ds(0, 32)], dst(col0 // 2, 32))


_relayout_sc = functools.partial(
    pl.kernel,
    out_type=jax.ShapeDtypeStruct((VOCAB // 2, PAIR), jnp.float32),
    mesh=plsc.VectorSubcoreMesh(core_axis_name="c", subcore_axis_name="s"),
    scratch_types=(
        [pltpu.VMEM((EMBED, BAND), jnp.float32)] * NBUF
        + [pltpu.VMEM((BAND // 2, PAIR), jnp.float32)] * NBUF
        + [pltpu.VMEM((EMBED, EMBED), jnp.float32)]
        + [pltpu.SemaphoreType.DMA] * (2 * NBUF)
    ),
    compiler_params=pltpu.CompilerParams(use_tc_tiling_on_sc=True,
                                         needs_layout_passes=False),
)(_relayout_body)


def _reduce_chunk(rows_ref, accs):
    def red(j, accs):
        sA = list(accs[0:4]); sB = list(accs[4:8])
        mA = list(accs[8:12]); mB = list(accs[12:16])
        for g in range(EGROUPS):
            va = rows_ref[2 * j, pl.ds(g * LANES, LANES)]
            vb = rows_ref[2 * j + 1, pl.ds(g * LANES, LANES)]
            sA[g] = sA[g] + va
            sB[g] = sB[g] + vb
            mA[g] = jnp.maximum(mA[g], va)
            mB[g] = jnp.maximum(mB[g], vb)
        return tuple(sA + sB + mA + mB)

    return lax.fori_loop(0, CHUNK // 2, red, accs, unroll=4)


def _fresh_accs():
    z = jnp.zeros((LANES,), jnp.float32)
    ninf = jnp.full((LANES,), -jnp.inf, jnp.float32)
    return tuple([z] * (2 * EGROUPS) + [ninf] * (2 * EGROUPS))


PBUF = 8


def _pool_body(idx_hbm, table_hbm, out_hbm, idx_v,
               rows0, rows1, rows2, rows3, rows4, rows5, rows6, rows7,
               pooled_v, sem0, sem1, sem2, sem3, sem4, sem5, sem6, sem7):
    wid = lax.axis_index("s") * NUM_CORES + lax.axis_index("c")
    base = wid * ROWS_PER_W
    rows = (rows0, rows1, rows2, rows3, rows4, rows5, rows6, rows7)
    sems = (sem0, sem1, sem2, sem3, sem4, sem5, sem6, sem7)

    pltpu.sync_copy(idx_hbm.at[pl.ds(wid * IDX_ROWS_PER_W, IDX_ROWS_PER_W)],
                    idx_v)

    def fire(c, b):
        pltpu.async_copy(table_hbm.at[idx_v.at[c]], rows[b], sems[b])

    def wait(b):
        pltpu.make_async_copy(table_hbm.at[idx_v.at[0]], rows[b],
                              sems[b]).wait()

    for b in range(PBUF):
        fire(b, b)

    def pair_body(ee, carry):
        for p in range(PBUF // 2):
            e = (PBUF // 2) * ee + p
            b0 = 2 * p
            b1 = 2 * p + 1
            wait(b0)
            accs = _reduce_chunk(rows[b0], _fresh_accs())
            fire(jnp.minimum(2 * e + PBUF, IDX_ROWS_PER_W - 1), b0)
            wait(b1)
            accs = _reduce_chunk(rows[b1], accs)
            fire(jnp.minimum(2 * e + PBUF + 1, IDX_ROWS_PER_W - 1), b1)

            inv_l = jnp.float32(1.0 / L)
            for g in range(EGROUPS):
                s = accs[g] + accs[EGROUPS + g]
                m = jnp.maximum(accs[2 * EGROUPS + g], accs[3 * EGROUPS + g])
                pooled_v[e, pl.ds(g * LANES, LANES)] = s * inv_l + m
        return carry

    lax.fori_loop(0, ROWS_PER_W // (PBUF // 2), pair_body, 0)
    for b in range(PBUF):
        wait(b)
    pltpu.sync_copy(pooled_v, out_hbm.at[pl.ds(base, ROWS_PER_W)])


_pool_sc = functools.partial(
    pl.kernel,
    out_type=jax.ShapeDtypeStruct((B, EMBED), jnp.float32),
    mesh=plsc.VectorSubcoreMesh(core_axis_name="c", subcore_axis_name="s"),
    scratch_types=(
        [pltpu.VMEM((IDX_ROWS_PER_W, CHUNK), jnp.int32)]
        + [pltpu.VMEM((CHUNK, EMBED), jnp.float32)] * PBUF
        + [pltpu.VMEM((ROWS_PER_W, EMBED), jnp.float32)]
        + [pltpu.SemaphoreType.DMA] * PBUF
    ),
    compiler_params=pltpu.CompilerParams(use_tc_tiling_on_sc=False,
                                         needs_layout_passes=False),
)(_pool_body)


def _mm_body(p_ref, w_ref, b_ref, o_ref):
    o_ref[...] = (
        jnp.dot(p_ref[...], w_ref[...], preferred_element_type=jnp.float32)
        + b_ref[...])


def _project(pooled, W, b):
    return pl.pallas_call(
        _mm_body,
        out_shape=jax.ShapeDtypeStruct((B, LABEL), jnp.float32),
    )(pooled, W, b.reshape(1, LABEL))


def kernel(input, emb_table, W, b):
    idx = input.astype(jnp.int32).reshape(B * L // CHUNK, CHUNK)
    emb_t = emb_table.T
    emb2 = _relayout_sc(emb_t, lax.slice(emb_t, (0, VOCAB - EMBED),
                                         (EMBED, VOCAB)))
    table_lin = emb2.reshape(VOCAB, EMBED)
    pooled = _pool_sc(idx, table_lin)
    return _project(pooled, W, b)
